# baseline (speedup 1.0000x reference)
import math

import numpy as np

import jax
import jax.numpy as jnp
from jax import lax
from jax.experimental import pallas as pl
from jax.experimental.pallas import tpu as pltpu

_NB = 8
_C = 32
_H = 8
_W = 32
_CIN = 4
_NH = 4
_HD = _W // _NH
_L = 2
_EPS = 1e-5
_CH = _C * _H


def _gelu(x):
    return 0.5 * x * (1.0 + lax.erf(x * (1.0 / math.sqrt(2.0))))


_S_SEL = np.zeros((3, _H, _H), np.float32)
for _dy in range(3):
    for _y in range(_H):
        _y2 = _y + _dy - 1
        if 0 <= _y2 < _H:
            _S_SEL[_dy, _y, _y2] = 1.0


def _shift_lanes(a, ox):
    if ox == 0:
        return a
    z = jnp.zeros(a.shape[:-1] + (abs(ox),), a.dtype)
    if ox > 0:
        return jnp.concatenate([a[..., ox:], z], axis=-1)
    return jnp.concatenate([z, a[..., :ox]], axis=-1)


def _encoder_kernel(x_ref, a1_ref, a2_ref, b1_ref, b2_ref, pos_ref,
                    ln1w_ref, ln1b_ref, wqkv_ref, bqkv_ref, wo_ref, bo_ref,
                    ln2w_ref, ln2b_ref, wm1_ref, bm1_ref, wm2_ref, bm2_ref,
                    o_ref):
    nb, ch, w = _NB, _CH, _W
    scale = 1.0 / math.sqrt(_HD)

    x2 = x_ref[...].reshape(nb * _CIN * _H, w).astype(jnp.bfloat16)
    x3 = jnp.concatenate([_shift_lanes(x2, -1), x2, _shift_lanes(x2, 1)],
                         axis=0)
    z1 = _gelu(jnp.dot(a1_ref[...], x3,
                       preferred_element_type=jnp.float32) + b1_ref[...])

    z1b = z1.astype(jnp.bfloat16)
    z1m = _shift_lanes(z1b, -1)
    z1p = _shift_lanes(z1b, 1)
    parts = []
    for n in range(nb):
        r = slice(n * ch, (n + 1) * ch)
        acc = (jnp.dot(a2_ref[0], z1m[r], preferred_element_type=jnp.float32)
               + jnp.dot(a2_ref[1], z1b[r], preferred_element_type=jnp.float32)
               + jnp.dot(a2_ref[2], z1p[r], preferred_element_type=jnp.float32))
        parts.append(acc)
    z2 = _gelu(jnp.concatenate(parts, axis=0) + b2_ref[...])

    x = z2 + pos_ref[...]

    r_id = lax.broadcasted_iota(jnp.int32, (1, w, w), 1)
    d_id = lax.broadcasted_iota(jnp.int32, (1, w, w), 2)
    hmask = (r_id // _HD) == (d_id // _HD)
    gsum = hmask[0].astype(jnp.float32)

    for l in range(_L):
        mu = jnp.mean(x, axis=-1, keepdims=True)
        var = jnp.mean(jnp.square(x - mu), axis=-1, keepdims=True)
        hln = (x - mu) * lax.rsqrt(var + _EPS) * ln1w_ref[l] + ln1b_ref[l]

        qkv = jnp.dot(hln.astype(jnp.bfloat16), wqkv_ref[l],
                      preferred_element_type=jnp.float32) + bqkv_ref[l]
        qkvb = qkv.astype(jnp.bfloat16)
        q3 = qkvb[:, 0 * w:1 * w].reshape(nb * _C, _H, w)
        k3 = qkvb[:, 1 * w:2 * w].reshape(nb * _C, _H, w)
        v3 = qkvb[:, 2 * w:3 * w].reshape(nb * _C, _H, w)

        bk = jnp.where(hmask, jnp.tile(k3, (1, _NH, 1)),
                       jnp.bfloat16(0.0))
        s = jnp.einsum("btd,bud->btu", q3, bk,
                       preferred_element_type=jnp.float32) * scale
        s2 = s.reshape(nb * ch, w)
        p = jnp.exp(s2 - jnp.max(s2, axis=-1, keepdims=True))
        denom = jnp.dot(p, gsum, preferred_element_type=jnp.float32)
        p3 = (p / denom).astype(jnp.bfloat16).reshape(nb * _C, _H, w)
        bv = jnp.where(hmask, jnp.tile(v3, (1, _NH, 1)),
                       jnp.bfloat16(0.0))
        att = jnp.einsum("btu,bud->btd", p3, bv,
                         preferred_element_type=jnp.float32)

        x = x + jnp.dot(att.reshape(nb * ch, w).astype(jnp.bfloat16),
                        wo_ref[l],
                        preferred_element_type=jnp.float32) + bo_ref[l]

        mu2 = jnp.mean(x, axis=-1, keepdims=True)
        var2 = jnp.mean(jnp.square(x - mu2), axis=-1, keepdims=True)
        h2 = (x - mu2) * lax.rsqrt(var2 + _EPS) * ln2w_ref[l] + ln2b_ref[l]
        m = _gelu(jnp.dot(h2.astype(jnp.bfloat16), wm1_ref[l],
                          preferred_element_type=jnp.float32) + bm1_ref[l])
        x = x + jnp.dot(m.astype(jnp.bfloat16), wm2_ref[l],
                        preferred_element_type=jnp.float32) + bm2_ref[l]

    o_ref[...] = x.reshape(nb, ch, w)


def kernel(x, conv1_w, conv1_b, conv2_w, conv2_b, pos, ln1w_0, ln1b_0, wq_0,
           bq_0, wk_0, bk_0, wv_0, bv_0, wo_0, bo_0, ln2w_0, ln2b_0, w1_0,
           b1_0, w2_0, b2_0, ln1w_1, ln1b_1, wq_1, bq_1, wk_1, bk_1, wv_1,
           bv_1, wo_1, bo_1, ln2w_1, ln2b_1, w1_1, b1_1, w2_1, b2_1):
    B = x.shape[0]
    nb, ch, w = _NB, _CH, _W
    ssel = jnp.asarray(_S_SEL)

    a1 = jnp.einsum("oidk,dyz->koyiz", conv1_w, ssel).reshape(3, ch, _CIN * _H)
    eye = jnp.eye(nb, dtype=jnp.float32)
    a1cat = jnp.concatenate([jnp.kron(eye, a1[k]) for k in range(3)],
                            axis=1).astype(jnp.bfloat16)
    a2 = jnp.einsum("oidk,dyz->koyiz", conv2_w,
                    ssel).reshape(3, ch, ch).astype(jnp.bfloat16)

    b1f = jnp.tile(jnp.broadcast_to(jnp.repeat(conv1_b, _H)[:, None],
                                    (ch, w)), (nb, 1))
    b2f = jnp.tile(jnp.broadcast_to(jnp.repeat(conv2_b, _H)[:, None],
                                    (ch, w)), (nb, 1))
    posf = jnp.tile(pos, (nb * _C, 1))

    blocks = [[ln1w_0, ln1b_0, wq_0, bq_0, wk_0, bk_0, wv_0, bv_0, wo_0, bo_0,
               ln2w_0, ln2b_0, w1_0, b1_0, w2_0, b2_0],
              [ln1w_1, ln1b_1, wq_1, bq_1, wk_1, bk_1, wv_1, bv_1, wo_1, bo_1,
               ln2w_1, ln2b_1, w1_1, b1_1, w2_1, b2_1]]
    ln1w = jnp.stack([b[0] for b in blocks]).reshape(_L, 1, _C)
    ln1b = jnp.stack([b[1] for b in blocks]).reshape(_L, 1, _C)
    wqkv = jnp.stack([jnp.concatenate([b[2], b[4], b[6]], axis=1)
                      for b in blocks]).astype(jnp.bfloat16)
    bqkv = jnp.stack([jnp.concatenate([b[3], b[5], b[7]])
                      for b in blocks]).reshape(_L, 1, 3 * _C)
    wo = jnp.stack([b[8] for b in blocks]).astype(jnp.bfloat16)
    bo = jnp.stack([b[9] for b in blocks]).reshape(_L, 1, _C)
    ln2w = jnp.stack([b[10] for b in blocks]).reshape(_L, 1, _C)
    ln2b = jnp.stack([b[11] for b in blocks]).reshape(_L, 1, _C)
    wm1 = jnp.stack([b[12] for b in blocks]).astype(jnp.bfloat16)
    bm1 = jnp.stack([b[13] for b in blocks]).reshape(_L, 1, 4 * _C)
    wm2 = jnp.stack([b[14] for b in blocks]).astype(jnp.bfloat16)
    bm2 = jnp.stack([b[15] for b in blocks]).reshape(_L, 1, _C)

    consts = [a1cat, a2, b1f, b2f, posf, ln1w, ln1b, wqkv, bqkv, wo, bo,
              ln2w, ln2b, wm1, bm1, wm2, bm2]

    def const_spec(a):
        nd = a.ndim
        return pl.BlockSpec(a.shape, lambda b, _nd=nd: (0,) * _nd)

    x_rows = x.reshape(B, _CIN * _H, w)

    flops = int(B * (2 * ch * 3 * _CIN * _H * w + 2 * ch * 3 * ch * w
                     + _L * (2 * ch * _C * 3 * _C
                             + 2 * _C * _NH * _H * _H * w * 2
                             + 2 * ch * _C * _C + 2 * ch * _C * 4 * _C * 2)))
    transc = int(2 * B * ch * w + _L * B * (_C * _NH * _H * _H + ch * 4 * _C))
    bytes_acc = int(4 * (2 * B * ch * w
                         + sum(int(np.prod(a.shape)) for a in consts)))

    out = pl.pallas_call(
        _encoder_kernel,
        grid=(B // nb,),
        in_specs=[pl.BlockSpec((nb, _CIN * _H, w), lambda b: (b, 0, 0))]
                 + [const_spec(a) for a in consts],
        out_specs=pl.BlockSpec((nb, ch, w), lambda b: (b, 0, 0)),
        out_shape=jax.ShapeDtypeStruct((B, ch, w), jnp.float32),
        compiler_params=pltpu.CompilerParams(dimension_semantics=("parallel",)),
        cost_estimate=pl.CostEstimate(flops=flops, transcendentals=transc,
                                      bytes_accessed=bytes_acc),
    )(x_rows, *consts)

    return out.reshape(B, _C, _H, _W)


# speedup vs baseline: 1.1135x; 1.1135x over previous
import math

import numpy as np

import jax
import jax.numpy as jnp
from jax import lax
from jax.experimental import pallas as pl
from jax.experimental.pallas import tpu as pltpu

_NB = 8
_C = 32
_H = 8
_W = 32
_CIN = 4
_NH = 4
_HD = _W // _NH
_L = 2
_EPS = 1e-5
_CH = _C * _H


def _gelu(x):
    return 0.5 * x * (1.0 + lax.erf(x * (1.0 / math.sqrt(2.0))))


_S_SEL = np.zeros((3, _H, _H), np.float32)
for _dy in range(3):
    for _y in range(_H):
        _y2 = _y + _dy - 1
        if 0 <= _y2 < _H:
            _S_SEL[_dy, _y, _y2] = 1.0


def _shift_groups(a, ox):
    if ox == 0:
        return a
    rolled = jnp.roll(a, -ox, axis=-1)
    lane = lax.broadcasted_iota(jnp.int32, a.shape, a.ndim - 1) % _W
    valid = (lane + ox >= 0) & (lane + ox < _W)
    return jnp.where(valid, rolled, 0.0)


def _encoder_kernel(x_ref, a1_ref, a2_ref, b1_ref, b2_ref, pos_ref,
                    ln1w_ref, ln1b_ref, wqkv_ref, bqkv_ref, wo_ref, bo_ref,
                    ln2w_ref, ln2b_ref, wm1_ref, bm1_ref, wm2_ref, bm2_ref,
                    o_ref):
    nb, ch, w = _NB, _CH, _W
    scale = 1.0 / math.sqrt(_HD)

    xt = x_ref[...]
    xw = jnp.concatenate([xt[n] for n in range(nb)], axis=1)
    acc1 = (jnp.dot(a1_ref[0], _shift_groups(xw, -1),
                    preferred_element_type=jnp.float32)
            + jnp.dot(a1_ref[1], xw, preferred_element_type=jnp.float32)
            + jnp.dot(a1_ref[2], _shift_groups(xw, 1),
                      preferred_element_type=jnp.float32))
    z1 = _gelu(acc1 + b1_ref[...])

    acc2 = (jnp.dot(a2_ref[0], _shift_groups(z1, -1),
                    preferred_element_type=jnp.float32)
            + jnp.dot(a2_ref[1], z1, preferred_element_type=jnp.float32)
            + jnp.dot(a2_ref[2], _shift_groups(z1, 1),
                      preferred_element_type=jnp.float32))
    z2w = _gelu(acc2 + b2_ref[...])

    z2 = jnp.concatenate([z2w[:, n * w:(n + 1) * w] for n in range(nb)],
                         axis=0)

    x = z2 + pos_ref[...]

    r_id = lax.broadcasted_iota(jnp.int32, (1, w, w), 1)
    d_id = lax.broadcasted_iota(jnp.int32, (1, w, w), 2)
    hmask = (r_id // _HD) == (d_id // _HD)
    gsum = hmask[0].astype(jnp.float32)

    for l in range(_L):
        mu = jnp.mean(x, axis=-1, keepdims=True)
        var = jnp.mean(jnp.square(x - mu), axis=-1, keepdims=True)
        hln = (x - mu) * lax.rsqrt(var + _EPS) * ln1w_ref[l] + ln1b_ref[l]

        qkv = jnp.dot(hln, wqkv_ref[l],
                      preferred_element_type=jnp.float32) + bqkv_ref[l]
        q3 = qkv[:, 0 * w:1 * w].reshape(nb * _C, _H, w)
        k3 = qkv[:, 1 * w:2 * w].reshape(nb * _C, _H, w)
        v3 = qkv[:, 2 * w:3 * w].reshape(nb * _C, _H, w)

        bk = jnp.where(hmask, jnp.tile(k3, (1, _NH, 1)), 0.0)
        s = jnp.einsum("btd,bud->btu", q3, bk,
                       preferred_element_type=jnp.float32) * scale
        s2 = s.reshape(nb * ch, w)
        p = jnp.exp(s2 - jnp.max(s2, axis=-1, keepdims=True))
        denom = jnp.dot(p, gsum, preferred_element_type=jnp.float32)
        p3 = (p / denom).reshape(nb * _C, _H, w)
        bv = jnp.where(hmask, jnp.tile(v3, (1, _NH, 1)), 0.0)
        att = jnp.einsum("btu,bud->btd", p3, bv,
                         preferred_element_type=jnp.float32)

        x = x + jnp.dot(att.reshape(nb * ch, w), wo_ref[l],
                        preferred_element_type=jnp.float32) + bo_ref[l]

        mu2 = jnp.mean(x, axis=-1, keepdims=True)
        var2 = jnp.mean(jnp.square(x - mu2), axis=-1, keepdims=True)
        h2 = (x - mu2) * lax.rsqrt(var2 + _EPS) * ln2w_ref[l] + ln2b_ref[l]
        m = _gelu(jnp.dot(h2, wm1_ref[l],
                          preferred_element_type=jnp.float32) + bm1_ref[l])
        x = x + jnp.dot(m, wm2_ref[l],
                        preferred_element_type=jnp.float32) + bm2_ref[l]

    o_ref[...] = x.reshape(nb, ch, w)


def kernel(x, conv1_w, conv1_b, conv2_w, conv2_b, pos, ln1w_0, ln1b_0, wq_0,
           bq_0, wk_0, bk_0, wv_0, bv_0, wo_0, bo_0, ln2w_0, ln2b_0, w1_0,
           b1_0, w2_0, b2_0, ln1w_1, ln1b_1, wq_1, bq_1, wk_1, bk_1, wv_1,
           bv_1, wo_1, bo_1, ln2w_1, ln2b_1, w1_1, b1_1, w2_1, b2_1):
    B = x.shape[0]
    nb, ch, w = _NB, _CH, _W
    ssel = jnp.asarray(_S_SEL)

    a1 = jnp.einsum("oidk,dyz->koyiz", conv1_w, ssel).reshape(3, ch, _CIN * _H)
    a2 = jnp.einsum("oidk,dyz->koyiz", conv2_w, ssel).reshape(3, ch, ch)

    b1c = jnp.repeat(conv1_b, _H)[:, None]
    b2c = jnp.repeat(conv2_b, _H)[:, None]
    posf = jnp.tile(pos, (nb * _C, 1))

    blocks = [[ln1w_0, ln1b_0, wq_0, bq_0, wk_0, bk_0, wv_0, bv_0, wo_0, bo_0,
               ln2w_0, ln2b_0, w1_0, b1_0, w2_0, b2_0],
              [ln1w_1, ln1b_1, wq_1, bq_1, wk_1, bk_1, wv_1, bv_1, wo_1, bo_1,
               ln2w_1, ln2b_1, w1_1, b1_1, w2_1, b2_1]]
    ln1w = jnp.stack([b[0] for b in blocks]).reshape(_L, 1, _C)
    ln1b = jnp.stack([b[1] for b in blocks]).reshape(_L, 1, _C)
    wqkv = jnp.stack([jnp.concatenate([b[2], b[4], b[6]], axis=1)
                      for b in blocks])
    bqkv = jnp.stack([jnp.concatenate([b[3], b[5], b[7]])
                      for b in blocks]).reshape(_L, 1, 3 * _C)
    wo = jnp.stack([b[8] for b in blocks])
    bo = jnp.stack([b[9] for b in blocks]).reshape(_L, 1, _C)
    ln2w = jnp.stack([b[10] for b in blocks]).reshape(_L, 1, _C)
    ln2b = jnp.stack([b[11] for b in blocks]).reshape(_L, 1, _C)
    wm1 = jnp.stack([b[12] for b in blocks])
    bm1 = jnp.stack([b[13] for b in blocks]).reshape(_L, 1, 4 * _C)
    wm2 = jnp.stack([b[14] for b in blocks])
    bm2 = jnp.stack([b[15] for b in blocks]).reshape(_L, 1, _C)

    consts = [a1, a2, b1c, b2c, posf, ln1w, ln1b, wqkv, bqkv, wo, bo,
              ln2w, ln2b, wm1, bm1, wm2, bm2]

    def const_spec(a):
        nd = a.ndim
        return pl.BlockSpec(a.shape, lambda b, _nd=nd: (0,) * _nd)

    x_rows = x.reshape(B, _CIN * _H, w)

    flops = int(B * (2 * ch * 3 * _CIN * _H * w + 2 * ch * 3 * ch * w
                     + _L * (2 * ch * _C * 3 * _C
                             + 2 * _C * _NH * _H * _H * w * 2
                             + 2 * ch * _C * _C + 2 * ch * _C * 4 * _C * 2)))
    transc = int(2 * B * ch * w + _L * B * (_C * _NH * _H * _H + ch * 4 * _C))
    bytes_acc = int(4 * (2 * B * ch * w
                         + sum(int(np.prod(a.shape)) for a in consts)))

    out = pl.pallas_call(
        _encoder_kernel,
        grid=(B // nb,),
        in_specs=[pl.BlockSpec((nb, _CIN * _H, w), lambda b: (b, 0, 0))]
                 + [const_spec(a) for a in consts],
        out_specs=pl.BlockSpec((nb, ch, w), lambda b: (b, 0, 0)),
        out_shape=jax.ShapeDtypeStruct((B, ch, w), jnp.float32),
        compiler_params=pltpu.CompilerParams(dimension_semantics=("parallel",)),
        cost_estimate=pl.CostEstimate(flops=flops, transcendentals=transc,
                                      bytes_accessed=bytes_acc),
    )(x_rows, *consts)

    return out.reshape(B, _C, _H, _W)


# speedup vs baseline: 1.4650x; 1.3156x over previous
import math

import numpy as np

import jax
import jax.numpy as jnp
from jax import lax
from jax.experimental import pallas as pl
from jax.experimental.pallas import tpu as pltpu

_NB = 16
_C = 32
_H = 8
_W = 32
_CIN = 4
_NH = 4
_HD = _W // _NH
_L = 2
_EPS = 1e-5
_CH = _C * _H


def _gelu(x):
    return 0.5 * x * (1.0 + lax.erf(x * (1.0 / math.sqrt(2.0))))


_S_SEL = np.zeros((3, _H, _H), np.float32)
for _dy in range(3):
    for _y in range(_H):
        _y2 = _y + _dy - 1
        if 0 <= _y2 < _H:
            _S_SEL[_dy, _y, _y2] = 1.0


def _shift_lanes(a, ox):
    if ox == 0:
        return a
    z = jnp.zeros(a.shape[:-1] + (abs(ox),), a.dtype)
    if ox > 0:
        return jnp.concatenate([a[..., ox:], z], axis=-1)
    return jnp.concatenate([z, a[..., :ox]], axis=-1)


def _encoder_kernel(x_ref, a1_ref, a2_ref, b1_ref, b2_ref, pos_ref,
                    ln1w_ref, ln1b_ref, wqkv_ref, bqkv_ref, wo_ref, bo_ref,
                    ln2w_ref, ln2b_ref, wm1_ref, bm1_ref, wm2_ref, bm2_ref,
                    o_ref):
    nb, ch, w = _NB, _CH, _W
    scale = 1.0 / math.sqrt(_HD)

    x2 = x_ref[...].reshape(nb * _CIN * _H, w)
    xm = _shift_lanes(x2, -1)
    xp = _shift_lanes(x2, 1)
    z1parts = []
    for n in range(nb):
        r = slice(n * _CIN * _H, (n + 1) * _CIN * _H)
        xn3 = jnp.concatenate([xm[r], x2[r], xp[r]], axis=0)
        z1parts.append(jnp.dot(a1_ref[...], xn3,
                               preferred_element_type=jnp.float32))
    z1 = _gelu(jnp.concatenate(z1parts, axis=0) + b1_ref[...])

    z1m = _shift_lanes(z1, -1)
    z1p = _shift_lanes(z1, 1)
    parts = []
    for n in range(nb):
        r = slice(n * ch, (n + 1) * ch)
        acc = (jnp.dot(a2_ref[0], z1m[r], preferred_element_type=jnp.float32)
               + jnp.dot(a2_ref[1], z1[r], preferred_element_type=jnp.float32)
               + jnp.dot(a2_ref[2], z1p[r], preferred_element_type=jnp.float32))
        parts.append(acc)
    z2 = _gelu(jnp.concatenate(parts, axis=0) + b2_ref[...])

    x = z2 + pos_ref[...]

    r_id = lax.broadcasted_iota(jnp.int32, (1, w, w), 1)
    d_id = lax.broadcasted_iota(jnp.int32, (1, w, w), 2)
    hmask = (r_id // _HD) == (d_id // _HD)
    gsum = hmask[0].astype(jnp.float32)

    for l in range(_L):
        mu = jnp.mean(x, axis=-1, keepdims=True)
        var = jnp.mean(jnp.square(x - mu), axis=-1, keepdims=True)
        hln = (x - mu) * lax.rsqrt(var + _EPS) * ln1w_ref[l] + ln1b_ref[l]

        qkv = jnp.dot(hln, wqkv_ref[l],
                      preferred_element_type=jnp.float32) + bqkv_ref[l]
        q3 = qkv[:, 0 * w:1 * w].reshape(nb * _C, _H, w)
        k3 = qkv[:, 1 * w:2 * w].reshape(nb * _C, _H, w)
        v3 = qkv[:, 2 * w:3 * w].reshape(nb * _C, _H, w)

        bk = jnp.where(hmask, jnp.tile(k3, (1, _NH, 1)), 0.0)
        s = jnp.einsum("btd,bud->btu", q3, bk,
                       preferred_element_type=jnp.float32) * scale
        s2 = s.reshape(nb * ch, w)
        p = jnp.exp(s2 - jnp.max(s2, axis=-1, keepdims=True))
        denom = jnp.dot(p, gsum, preferred_element_type=jnp.float32)
        p3 = (p / denom).reshape(nb * _C, _H, w)
        bv = jnp.where(hmask, jnp.tile(v3, (1, _NH, 1)), 0.0)
        att = jnp.einsum("btu,bud->btd", p3, bv,
                         preferred_element_type=jnp.float32)

        x = x + jnp.dot(att.reshape(nb * ch, w), wo_ref[l],
                        preferred_element_type=jnp.float32) + bo_ref[l]

        mu2 = jnp.mean(x, axis=-1, keepdims=True)
        var2 = jnp.mean(jnp.square(x - mu2), axis=-1, keepdims=True)
        h2 = (x - mu2) * lax.rsqrt(var2 + _EPS) * ln2w_ref[l] + ln2b_ref[l]
        m = _gelu(jnp.dot(h2, wm1_ref[l],
                          preferred_element_type=jnp.float32) + bm1_ref[l])
        x = x + jnp.dot(m, wm2_ref[l],
                        preferred_element_type=jnp.float32) + bm2_ref[l]

    o_ref[...] = x.reshape(nb, ch, w)


def kernel(x, conv1_w, conv1_b, conv2_w, conv2_b, pos, ln1w_0, ln1b_0, wq_0,
           bq_0, wk_0, bk_0, wv_0, bv_0, wo_0, bo_0, ln2w_0, ln2b_0, w1_0,
           b1_0, w2_0, b2_0, ln1w_1, ln1b_1, wq_1, bq_1, wk_1, bk_1, wv_1,
           bv_1, wo_1, bo_1, ln2w_1, ln2b_1, w1_1, b1_1, w2_1, b2_1):
    B = x.shape[0]
    nb, ch, w = _NB, _CH, _W
    ssel = jnp.asarray(_S_SEL)

    a1 = jnp.einsum("oidk,dyz->koyiz", conv1_w, ssel).reshape(3, ch, _CIN * _H)
    a1cat = jnp.concatenate([a1[0], a1[1], a1[2]], axis=1)
    a2 = jnp.einsum("oidk,dyz->koyiz", conv2_w, ssel).reshape(3, ch, ch)

    b1f = jnp.tile(jnp.broadcast_to(jnp.repeat(conv1_b, _H)[:, None],
                                    (ch, w)), (nb, 1))
    b2f = jnp.tile(jnp.broadcast_to(jnp.repeat(conv2_b, _H)[:, None],
                                    (ch, w)), (nb, 1))
    posf = jnp.tile(pos, (nb * _C, 1))

    blocks = [[ln1w_0, ln1b_0, wq_0, bq_0, wk_0, bk_0, wv_0, bv_0, wo_0, bo_0,
               ln2w_0, ln2b_0, w1_0, b1_0, w2_0, b2_0],
              [ln1w_1, ln1b_1, wq_1, bq_1, wk_1, bk_1, wv_1, bv_1, wo_1, bo_1,
               ln2w_1, ln2b_1, w1_1, b1_1, w2_1, b2_1]]
    ln1w = jnp.stack([b[0] for b in blocks]).reshape(_L, 1, _C)
    ln1b = jnp.stack([b[1] for b in blocks]).reshape(_L, 1, _C)
    wqkv = jnp.stack([jnp.concatenate([b[2], b[4], b[6]], axis=1)
                      for b in blocks])
    bqkv = jnp.stack([jnp.concatenate([b[3], b[5], b[7]])
                      for b in blocks]).reshape(_L, 1, 3 * _C)
    wo = jnp.stack([b[8] for b in blocks])
    bo = jnp.stack([b[9] for b in blocks]).reshape(_L, 1, _C)
    ln2w = jnp.stack([b[10] for b in blocks]).reshape(_L, 1, _C)
    ln2b = jnp.stack([b[11] for b in blocks]).reshape(_L, 1, _C)
    wm1 = jnp.stack([b[12] for b in blocks])
    bm1 = jnp.stack([b[13] for b in blocks]).reshape(_L, 1, 4 * _C)
    wm2 = jnp.stack([b[14] for b in blocks])
    bm2 = jnp.stack([b[15] for b in blocks]).reshape(_L, 1, _C)

    consts = [a1cat, a2, b1f, b2f, posf, ln1w, ln1b, wqkv, bqkv, wo, bo,
              ln2w, ln2b, wm1, bm1, wm2, bm2]

    def const_spec(a):
        nd = a.ndim
        return pl.BlockSpec(a.shape, lambda b, _nd=nd: (0,) * _nd)

    x_rows = x.reshape(B, _CIN * _H, w)

    flops = int(B * (2 * ch * 3 * _CIN * _H * w + 2 * ch * 3 * ch * w
                     + _L * (2 * ch * _C * 3 * _C
                             + 2 * _C * _NH * _H * _H * w * 2
                             + 2 * ch * _C * _C + 2 * ch * _C * 4 * _C * 2)))
    transc = int(2 * B * ch * w + _L * B * (_C * _NH * _H * _H + ch * 4 * _C))
    bytes_acc = int(4 * (2 * B * ch * w
                         + sum(int(np.prod(a.shape)) for a in consts)))

    out = pl.pallas_call(
        _encoder_kernel,
        grid=(B // nb,),
        in_specs=[pl.BlockSpec((nb, _CIN * _H, w), lambda b: (b, 0, 0))]
                 + [const_spec(a) for a in consts],
        out_specs=pl.BlockSpec((nb, ch, w), lambda b: (b, 0, 0)),
        out_shape=jax.ShapeDtypeStruct((B, ch, w), jnp.float32),
        compiler_params=pltpu.CompilerParams(dimension_semantics=("parallel",)),
        cost_estimate=pl.CostEstimate(flops=flops, transcendentals=transc,
                                      bytes_accessed=bytes_acc),
    )(x_rows, *consts)

    return out.reshape(B, _C, _H, _W)


# speedup vs baseline: 1.4725x; 1.0052x over previous
import math

import numpy as np

import jax
import jax.numpy as jnp
from jax import lax
from jax.experimental import pallas as pl
from jax.experimental.pallas import tpu as pltpu

_NB = 32
_C = 32
_H = 8
_W = 32
_CIN = 4
_NH = 4
_HD = _W // _NH
_L = 2
_EPS = 1e-5
_CH = _C * _H


def _gelu(x):
    return 0.5 * x * (1.0 + lax.erf(x * (1.0 / math.sqrt(2.0))))


_S_SEL = np.zeros((3, _H, _H), np.float32)
for _dy in range(3):
    for _y in range(_H):
        _y2 = _y + _dy - 1
        if 0 <= _y2 < _H:
            _S_SEL[_dy, _y, _y2] = 1.0


def _shift_lanes(a, ox):
    if ox == 0:
        return a
    z = jnp.zeros(a.shape[:-1] + (abs(ox),), a.dtype)
    if ox > 0:
        return jnp.concatenate([a[..., ox:], z], axis=-1)
    return jnp.concatenate([z, a[..., :ox]], axis=-1)


def _encoder_kernel(x_ref, a1_ref, a2_ref, b1_ref, b2_ref, pos_ref,
                    ln1w_ref, ln1b_ref, wqkv_ref, bqkv_ref, wo_ref, bo_ref,
                    ln2w_ref, ln2b_ref, wm1_ref, bm1_ref, wm2_ref, bm2_ref,
                    o_ref):
    nb, ch, w = _NB, _CH, _W
    scale = 1.0 / math.sqrt(_HD)

    x2 = x_ref[...].reshape(nb * _CIN * _H, w)
    xm = _shift_lanes(x2, -1)
    xp = _shift_lanes(x2, 1)
    z1parts = []
    for n in range(nb):
        r = slice(n * _CIN * _H, (n + 1) * _CIN * _H)
        xn3 = jnp.concatenate([xm[r], x2[r], xp[r]], axis=0)
        z1parts.append(jnp.dot(a1_ref[...], xn3,
                               preferred_element_type=jnp.float32))
    z1 = _gelu(jnp.concatenate(z1parts, axis=0) + b1_ref[...])

    z1m = _shift_lanes(z1, -1)
    z1p = _shift_lanes(z1, 1)
    parts = []
    for n in range(nb):
        r = slice(n * ch, (n + 1) * ch)
        acc = (jnp.dot(a2_ref[0], z1m[r], preferred_element_type=jnp.float32)
               + jnp.dot(a2_ref[1], z1[r], preferred_element_type=jnp.float32)
               + jnp.dot(a2_ref[2], z1p[r], preferred_element_type=jnp.float32))
        parts.append(acc)
    z2 = _gelu(jnp.concatenate(parts, axis=0) + b2_ref[...])

    x = z2 + pos_ref[...]

    r_id = lax.broadcasted_iota(jnp.int32, (1, w, w), 1)
    d_id = lax.broadcasted_iota(jnp.int32, (1, w, w), 2)
    hmask = (r_id // _HD) == (d_id // _HD)
    gsum = hmask[0].astype(jnp.float32)

    for l in range(_L):
        mu = jnp.mean(x, axis=-1, keepdims=True)
        var = jnp.mean(jnp.square(x - mu), axis=-1, keepdims=True)
        hln = (x - mu) * lax.rsqrt(var + _EPS) * ln1w_ref[l] + ln1b_ref[l]

        qkv = jnp.dot(hln, wqkv_ref[l],
                      preferred_element_type=jnp.float32) + bqkv_ref[l]
        q3 = qkv[:, 0 * w:1 * w].reshape(nb * _C, _H, w)
        k3 = qkv[:, 1 * w:2 * w].reshape(nb * _C, _H, w)
        v3 = qkv[:, 2 * w:3 * w].reshape(nb * _C, _H, w)

        bk = jnp.where(hmask, jnp.tile(k3, (1, _NH, 1)), 0.0)
        s = jnp.einsum("btd,bud->btu", q3, bk,
                       preferred_element_type=jnp.float32) * scale
        s2 = s.reshape(nb * ch, w)
        p = jnp.exp(s2 - jnp.max(s2, axis=-1, keepdims=True))
        denom = jnp.dot(p, gsum, preferred_element_type=jnp.float32)
        p3 = (p / denom).reshape(nb * _C, _H, w)
        bv = jnp.where(hmask, jnp.tile(v3, (1, _NH, 1)), 0.0)
        att = jnp.einsum("btu,bud->btd", p3, bv,
                         preferred_element_type=jnp.float32)

        x = x + jnp.dot(att.reshape(nb * ch, w), wo_ref[l],
                        preferred_element_type=jnp.float32) + bo_ref[l]

        mu2 = jnp.mean(x, axis=-1, keepdims=True)
        var2 = jnp.mean(jnp.square(x - mu2), axis=-1, keepdims=True)
        h2 = (x - mu2) * lax.rsqrt(var2 + _EPS) * ln2w_ref[l] + ln2b_ref[l]
        m = _gelu(jnp.dot(h2, wm1_ref[l],
                          preferred_element_type=jnp.float32) + bm1_ref[l])
        x = x + jnp.dot(m, wm2_ref[l],
                        preferred_element_type=jnp.float32) + bm2_ref[l]

    o_ref[...] = x.reshape(nb, ch, w)


def kernel(x, conv1_w, conv1_b, conv2_w, conv2_b, pos, ln1w_0, ln1b_0, wq_0,
           bq_0, wk_0, bk_0, wv_0, bv_0, wo_0, bo_0, ln2w_0, ln2b_0, w1_0,
           b1_0, w2_0, b2_0, ln1w_1, ln1b_1, wq_1, bq_1, wk_1, bk_1, wv_1,
           bv_1, wo_1, bo_1, ln2w_1, ln2b_1, w1_1, b1_1, w2_1, b2_1):
    B = x.shape[0]
    nb, ch, w = _NB, _CH, _W
    ssel = jnp.asarray(_S_SEL)

    a1 = jnp.einsum("oidk,dyz->koyiz", conv1_w, ssel).reshape(3, ch, _CIN * _H)
    a1cat = jnp.concatenate([a1[0], a1[1], a1[2]], axis=1)
    a2 = jnp.einsum("oidk,dyz->koyiz", conv2_w, ssel).reshape(3, ch, ch)

    b1f = jnp.tile(jnp.broadcast_to(jnp.repeat(conv1_b, _H)[:, None],
                                    (ch, w)), (nb, 1))
    b2f = jnp.tile(jnp.broadcast_to(jnp.repeat(conv2_b, _H)[:, None],
                                    (ch, w)), (nb, 1))
    posf = jnp.tile(pos, (nb * _C, 1))

    blocks = [[ln1w_0, ln1b_0, wq_0, bq_0, wk_0, bk_0, wv_0, bv_0, wo_0, bo_0,
               ln2w_0, ln2b_0, w1_0, b1_0, w2_0, b2_0],
              [ln1w_1, ln1b_1, wq_1, bq_1, wk_1, bk_1, wv_1, bv_1, wo_1, bo_1,
               ln2w_1, ln2b_1, w1_1, b1_1, w2_1, b2_1]]
    ln1w = jnp.stack([b[0] for b in blocks]).reshape(_L, 1, _C)
    ln1b = jnp.stack([b[1] for b in blocks]).reshape(_L, 1, _C)
    wqkv = jnp.stack([jnp.concatenate([b[2], b[4], b[6]], axis=1)
                      for b in blocks])
    bqkv = jnp.stack([jnp.concatenate([b[3], b[5], b[7]])
                      for b in blocks]).reshape(_L, 1, 3 * _C)
    wo = jnp.stack([b[8] for b in blocks])
    bo = jnp.stack([b[9] for b in blocks]).reshape(_L, 1, _C)
    ln2w = jnp.stack([b[10] for b in blocks]).reshape(_L, 1, _C)
    ln2b = jnp.stack([b[11] for b in blocks]).reshape(_L, 1, _C)
    wm1 = jnp.stack([b[12] for b in blocks])
    bm1 = jnp.stack([b[13] for b in blocks]).reshape(_L, 1, 4 * _C)
    wm2 = jnp.stack([b[14] for b in blocks])
    bm2 = jnp.stack([b[15] for b in blocks]).reshape(_L, 1, _C)

    consts = [a1cat, a2, b1f, b2f, posf, ln1w, ln1b, wqkv, bqkv, wo, bo,
              ln2w, ln2b, wm1, bm1, wm2, bm2]

    def const_spec(a):
        nd = a.ndim
        return pl.BlockSpec(a.shape, lambda b, _nd=nd: (0,) * _nd)

    x_rows = x.reshape(B, _CIN * _H, w)

    flops = int(B * (2 * ch * 3 * _CIN * _H * w + 2 * ch * 3 * ch * w
                     + _L * (2 * ch * _C * 3 * _C
                             + 2 * _C * _NH * _H * _H * w * 2
                             + 2 * ch * _C * _C + 2 * ch * _C * 4 * _C * 2)))
    transc = int(2 * B * ch * w + _L * B * (_C * _NH * _H * _H + ch * 4 * _C))
    bytes_acc = int(4 * (2 * B * ch * w
                         + sum(int(np.prod(a.shape)) for a in consts)))

    out = pl.pallas_call(
        _encoder_kernel,
        grid=(B // nb,),
        in_specs=[pl.BlockSpec((nb, _CIN * _H, w), lambda b: (b, 0, 0))]
                 + [const_spec(a) for a in consts],
        out_specs=pl.BlockSpec((nb, ch, w), lambda b: (b, 0, 0)),
        out_shape=jax.ShapeDtypeStruct((B, ch, w), jnp.float32),
        compiler_params=pltpu.CompilerParams(dimension_semantics=("parallel",)),
        cost_estimate=pl.CostEstimate(flops=flops, transcendentals=transc,
                                      bytes_accessed=bytes_acc),
    )(x_rows, *consts)

    return out.reshape(B, _C, _H, _W)
